# baseline (speedup 1.0000x reference)
import functools

import jax
import jax.numpy as jnp
from jax import lax
from jax.experimental import pallas as pl
from jax.experimental.pallas import tpu as pltpu
from jax.experimental.pallas import tpu_sc as plsc

NC = 2
NS = 16
NW = NC * NS

CHUNK = 128
NBUF = 5

PROJ_BLK = 32768
WIDEN_BLK = 8192


def _project_table(tT, W, b2):
    D, V = tT.shape
    H = W.shape[1]
    Hh = H // 2
    grid = (pl.cdiv(V, PROJ_BLK),)

    def body(t_ref, w_ref, b_ref, p_ref):
        acc = lax.dot_general(
            t_ref[...], w_ref[...],
            dimension_numbers=(((0,), (0,)), ((), ())),
            preferred_element_type=jnp.float32) + b_ref[...]
        hb = acc[:, :Hh].astype(jnp.bfloat16)
        lb = acc[:, Hh:].astype(jnp.bfloat16)
        hu = lax.bitcast_convert_type(hb, jnp.uint16).astype(jnp.uint32)
        lu = lax.bitcast_convert_type(lb, jnp.uint16).astype(jnp.uint32)
        p_ref[...] = lax.bitcast_convert_type((hu << 16) | lu, jnp.int32)

    return pl.pallas_call(
        body,
        grid=grid,
        in_specs=[
            pl.BlockSpec((D, PROJ_BLK), lambda i: (0, i)),
            pl.BlockSpec(W.shape, lambda i: (0, 0)),
            pl.BlockSpec((1, H), lambda i: (0, 0)),
        ],
        out_specs=pl.BlockSpec((PROJ_BLK, Hh), lambda i: (i, 0)),
        out_shape=jax.ShapeDtypeStruct((V, Hh), jnp.int32),
    )(tT, W, b2)


def _widen(x, H):
    Bs, Hh = x.shape
    grid = (Bs // WIDEN_BLK,)

    def body(x_ref, o_ref):
        u = lax.bitcast_convert_type(x_ref[...], jnp.uint32)
        hi = (u >> 16).astype(jnp.uint16)
        lo = u.astype(jnp.uint16)
        hf = lax.bitcast_convert_type(hi, jnp.bfloat16).astype(jnp.float32)
        lf = lax.bitcast_convert_type(lo, jnp.bfloat16).astype(jnp.float32)
        o_ref[...] = jnp.concatenate([hf, lf], axis=1)

    return pl.pallas_call(
        body,
        grid=grid,
        in_specs=[pl.BlockSpec((WIDEN_BLK, Hh), lambda i: (i, 0))],
        out_specs=pl.BlockSpec((WIDEN_BLK, H), lambda i: (i, 0)),
        out_shape=jax.ShapeDtypeStruct((Bs, H), jnp.float32),
    )(x)


def _make_gather(Bs, Hh):
    bpw = Bs // NW
    cpw = bpw // CHUNK
    outer_iters = cpw // NBUF
    mesh = plsc.VectorSubcoreMesh(
        core_axis_name="c", subcore_axis_name="s",
        num_cores=NC, num_subcores=NS)

    @functools.partial(
        pl.kernel,
        mesh=mesh,
        compiler_params=pltpu.CompilerParams(use_tc_tiling_on_sc=False),
        out_type=jax.ShapeDtypeStruct((Bs, Hh), jnp.int32),
        scratch_types=[
            pltpu.VMEM((cpw, CHUNK), jnp.int32),
            pltpu.VMEM((NBUF, CHUNK, Hh), jnp.int32),
        ] + [pltpu.SemaphoreType.DMA] * (2 * NBUF),
    )
    def gather_kernel(gidx_hbm, p_hbm, out_hbm, idx_v, rows_v, *sems):
        gsem, osem = sems[:NBUF], sems[NBUF:]
        wid = lax.axis_index("s") * NC + lax.axis_index("c")
        base_row = wid * bpw
        pltpu.sync_copy(gidx_hbm.at[pl.ds(wid * cpw, cpw)], idx_v)

        def gfire(i, b):
            pltpu.async_copy(p_hbm.at[idx_v.at[i]], rows_v.at[b], gsem[b])

        def gwait(b):
            pltpu.make_async_copy(p_hbm.at[pl.ds(0, CHUNK)],
                                  rows_v.at[b], gsem[b]).wait()

        def ofire(i, b):
            pltpu.async_copy(rows_v.at[b],
                             out_hbm.at[pl.ds(base_row + i * CHUNK, CHUNK)],
                             osem[b])

        def owait(b):
            pltpu.make_async_copy(rows_v.at[b],
                                  out_hbm.at[pl.ds(0, CHUNK)], osem[b]).wait()

        for b in range(NBUF - 1):
            gfire(b, b)
        for b in range(NBUF):
            gwait(b)
            ofire(b, b)
            bb = (b + NBUF - 1) % NBUF
            if b >= 1:
                owait(bb)
            gfire(b + NBUF - 1, bb)

        def outer(oo, carry):
            for b in range(NBUF):
                i = oo * NBUF + b
                gwait(b)
                ofire(i, b)
                bb = (b + NBUF - 1) % NBUF
                owait(bb)
                gfire(i + NBUF - 1, bb)
            return carry

        lax.fori_loop(1, outer_iters - 1, outer, 0)
        for b in range(NBUF):
            i = (outer_iters - 1) * NBUF + b
            gwait(b)
            ofire(i, b)
            if b == 0:
                bb = (b + NBUF - 1) % NBUF
                owait(bb)
                gfire(i + NBUF - 1, bb)
        for b in range(NBUF):
            owait(b)

    return gather_kernel


def kernel(sent1, sent2, table, W, b):
    batch, seq = sent1.shape
    V, D = table.shape
    H = W.shape[1]
    P32 = _project_table(table.T, W, b.reshape(1, H))

    gather = _make_gather(batch * seq, H // 2)

    def one(sent):
        g = sent.transpose(1, 0).reshape(NW * (batch * seq) // (NW * CHUNK),
                                         CHUNK)
        out = _widen(gather(g, P32), H)
        return out.reshape(seq, batch, H).transpose(1, 0, 2)

    return one(sent1), one(sent2)


# speedup vs baseline: 2.5136x; 2.5136x over previous
import functools

import jax
import jax.numpy as jnp
from jax import lax
from jax.experimental import pallas as pl
from jax.experimental.pallas import tpu as pltpu
from jax.experimental.pallas import tpu_sc as plsc

NC = 2
NS = 16
NW = NC * NS

CHUNK = 128
NBUF = 5

PROJ_BLK = 32768


def _project_table(tT, W, b2):
    D, V = tT.shape
    H = W.shape[1]
    grid = (pl.cdiv(V, PROJ_BLK),)

    def body(t_ref, w_ref, b_ref, p_ref):
        p_ref[...] = lax.dot_general(
            t_ref[...], w_ref[...],
            dimension_numbers=(((0,), (0,)), ((), ())),
            preferred_element_type=jnp.float32) + b_ref[...]

    return pl.pallas_call(
        body,
        grid=grid,
        in_specs=[
            pl.BlockSpec((D, PROJ_BLK), lambda i: (0, i)),
            pl.BlockSpec(W.shape, lambda i: (0, 0)),
            pl.BlockSpec((1, H), lambda i: (0, 0)),
        ],
        out_specs=pl.BlockSpec((PROJ_BLK, H), lambda i: (i, 0)),
        out_shape=jax.ShapeDtypeStruct((V, H), jnp.float32),
    )(tT, W, b2)


def _make_gather(Bs, H):
    bpw = Bs // NW
    cpw = bpw // CHUNK
    outer_iters = cpw // NBUF
    mesh = plsc.VectorSubcoreMesh(
        core_axis_name="c", subcore_axis_name="s",
        num_cores=NC, num_subcores=NS)

    @functools.partial(
        pl.kernel,
        mesh=mesh,
        out_type=jax.ShapeDtypeStruct((Bs, H), jnp.float32),
        scratch_types=[
            pltpu.VMEM((cpw, CHUNK), jnp.int32),
            pltpu.VMEM((NBUF, CHUNK, H), jnp.float32),
        ] + [pltpu.SemaphoreType.DMA] * (2 * NBUF),
    )
    def gather_kernel(gidx_hbm, p_hbm, out_hbm, idx_v, rows_v, *sems):
        gsem, osem = sems[:NBUF], sems[NBUF:]
        wid = lax.axis_index("s") * NC + lax.axis_index("c")
        base_row = wid * bpw
        pltpu.sync_copy(gidx_hbm.at[wid], idx_v)

        def gfire(i, b):
            pltpu.async_copy(p_hbm.at[idx_v.at[i]], rows_v.at[b], gsem[b])

        def gwait(b):
            pltpu.make_async_copy(out_hbm.at[pl.ds(0, CHUNK)],
                                  rows_v.at[b], gsem[b]).wait()

        def ofire(i, b):
            pltpu.async_copy(rows_v.at[b],
                             out_hbm.at[pl.ds(base_row + i * CHUNK, CHUNK)],
                             osem[b])

        def owait(b):
            pltpu.make_async_copy(rows_v.at[b],
                                  out_hbm.at[pl.ds(0, CHUNK)], osem[b]).wait()

        for b in range(NBUF - 1):
            gfire(b, b)
        for b in range(NBUF):
            gwait(b)
            ofire(b, b)
            bb = (b + NBUF - 1) % NBUF
            if b >= 1:
                owait(bb)
            gfire(b + NBUF - 1, bb)

        def outer(oo, carry):
            for b in range(NBUF):
                i = oo * NBUF + b
                gwait(b)
                ofire(i, b)
                bb = (b + NBUF - 1) % NBUF
                owait(bb)
                gfire(i + NBUF - 1, bb)
            return carry

        lax.fori_loop(1, outer_iters - 1, outer, 0)
        for b in range(NBUF):
            i = (outer_iters - 1) * NBUF + b
            gwait(b)
            ofire(i, b)
            if b == 0:
                bb = (b + NBUF - 1) % NBUF
                owait(bb)
                gfire(i + NBUF - 1, bb)
        for b in range(NBUF):
            owait(b)

    return gather_kernel


def kernel(sent1, sent2, table, W, b):
    batch, seq = sent1.shape
    V, D = table.shape
    H = W.shape[1]
    P = _project_table(table.T, W, b.reshape(1, H))

    gather = _make_gather(batch * seq, H)

    def one(sent):
        g = sent.transpose(1, 0).reshape(NW, (batch * seq) // (NW * CHUNK),
                                         CHUNK)
        out = gather(g, P)
        return out.reshape(seq, batch, H).transpose(1, 0, 2)

    return one(sent1), one(sent2)
